# baseline (speedup 1.0000x reference)
import functools

import jax
import jax.numpy as jnp
from jax.experimental import pallas as pl
from jax.experimental.pallas import tpu as pltpu

MBLK = 64
NEG = -1e37


def _attn_kernel(q_ref, k_ref, v_ref, m_ref, o_ref, allow_ref, *, qt, tk):
    qi = pl.program_id(1)
    S = k_ref.shape[2]
    D = k_ref.shape[3]
    nb = S // MBLK
    ng = qt // MBLK

    q = q_ref[0, 0]
    mask_slab = m_ref[0].astype(jnp.float32)

    g_rows = jax.lax.broadcasted_iota(jnp.int32, (ng, qt), 0)
    g_cols = jax.lax.broadcasted_iota(jnp.int32, (ng, qt), 1)
    G = (g_cols // MBLK == g_rows).astype(jnp.float32)
    colsum = jax.lax.dot_general(
        G, mask_slab, (((1,), (0,)), ((), ())),
        preferred_element_type=jnp.float32)
    blk_ids = jax.lax.broadcasted_iota(jnp.int32, (nb, S), 0)
    col_ids = jax.lax.broadcasted_iota(jnp.int32, (nb, S), 1)
    E = (col_ids // MBLK == blk_ids).astype(jnp.float32)
    blocksum = jax.lax.dot_general(
        colsum, E, (((1,), (1,)), ((), ())),
        preferred_element_type=jnp.float32)
    active = (blocksum > 0).astype(jnp.float32)
    allow_ref[...] = jax.lax.dot_general(
        active, E, (((1,), (0,)), ((), ())),
        preferred_element_type=jnp.float32)

    row_ids = qi * qt + jax.lax.broadcasted_iota(jnp.int32, (qt, tk), 0)
    col_iota = jax.lax.broadcasted_iota(jnp.int32, (qt, tk), 1)
    n_chunks = qi * qt // tk + 1
    m0 = jnp.full((qt, 1), NEG, jnp.float32)
    l0 = jnp.zeros((qt, 1), jnp.float32)
    acc0 = jnp.zeros((qt, D), jnp.float32)

    def qk(j):
        k = k_ref[0, 0, pl.ds(j * tk, tk), :]
        return jax.lax.dot_general(
            q, k, (((1,), (1,)), ((), ())),
            preferred_element_type=jnp.float32)

    def flash_update(j, s, carry):
        m_i, l_i, acc = carry
        v = v_ref[0, 0, pl.ds(j * tk, tk), :]
        m_new = jnp.maximum(m_i, jnp.max(s, axis=1, keepdims=True))
        p = jnp.exp(s - m_new)
        alpha = jnp.exp(m_i - m_new)
        l_new = l_i * alpha + jnp.sum(p, axis=1, keepdims=True)
        acc_new = acc * alpha + jax.lax.dot_general(
            p, v, (((1,), (0,)), ((), ())),
            preferred_element_type=jnp.float32)
        return m_new, l_new, acc_new

    all_active = jnp.min(blocksum) > 0

    @pl.when(all_active)
    def _fast():
        def body(j, carry):
            return flash_update(j, qk(j), carry)

        carry = jax.lax.fori_loop(0, n_chunks - 1, body, (m0, l0, acc0))
        jd = n_chunks - 1
        s = qk(jd)
        s = jnp.where(jd * tk + col_iota <= row_ids, s, NEG)
        m_f, l_f, acc_f = flash_update(jd, s, carry)
        o_ref[0, 0] = acc_f / l_f

    @pl.when(jnp.logical_not(all_active))
    def _general():
        def body(j, carry):
            m_i, l_i, acc = carry
            s = qk(j)
            v = v_ref[0, 0, pl.ds(j * tk, tk), :]
            allow_g = allow_ref[:, pl.ds(j * tk, tk)]
            ballow = jnp.concatenate(
                [jnp.broadcast_to(allow_g[g:g + 1, :], (MBLK, tk))
                 for g in range(ng)], axis=0) > 0.5
            allow = ballow & (j * tk + col_iota <= row_ids)
            s = jnp.where(allow, s, NEG)
            m_new = jnp.maximum(m_i, jnp.max(s, axis=1, keepdims=True))
            p = jnp.exp(s - m_new) * allow.astype(jnp.float32)
            alpha = jnp.exp(m_i - m_new)
            l_new = l_i * alpha + jnp.sum(p, axis=1, keepdims=True)
            acc_new = acc * alpha + jax.lax.dot_general(
                p, v, (((1,), (0,)), ((), ())),
                preferred_element_type=jnp.float32)
            return m_new, l_new, acc_new

        m_f, l_f, acc_f = jax.lax.fori_loop(0, n_chunks, body, (m0, l0, acc0))
        o_ref[0, 0] = jnp.where(l_f > 0, acc_f / jnp.maximum(l_f, 1e-30), 0.0)


@jax.jit
def kernel(query, key, value, mask):
    B, H, S, D = query.shape
    qt = min(512, S)
    tk = min(2048, S)
    ng = qt // MBLK
    grid = (H, S // qt)
    out = pl.pallas_call(
        functools.partial(_attn_kernel, qt=qt, tk=tk),
        grid=grid,
        in_specs=[
            pl.BlockSpec((1, 1, qt, D), lambda h, i: (0, h, i, 0)),
            pl.BlockSpec((1, 1, S, D), lambda h, i: (0, h, 0, 0)),
            pl.BlockSpec((1, 1, S, D), lambda h, i: (0, h, 0, 0)),
            pl.BlockSpec((1, qt, S), lambda h, i: (h, i, 0)),
        ],
        out_specs=pl.BlockSpec((1, 1, qt, D), lambda h, i: (0, h, i, 0)),
        out_shape=jax.ShapeDtypeStruct((B, H, S, D), jnp.float32),
        scratch_shapes=[pltpu.VMEM((ng, S), jnp.float32)],
        compiler_params=pltpu.CompilerParams(
            dimension_semantics=("parallel", "arbitrary")),
    )(query, key, value, mask)
    return out


# speedup vs baseline: 1.2544x; 1.2544x over previous
import functools

import jax
import jax.numpy as jnp
from jax.experimental import pallas as pl
from jax.experimental.pallas import tpu as pltpu

MBLK = 64
NEG = -1e37


def _attn_kernel(q_ref, k_ref, v_ref, m_ref, o_ref, allow_ref, *, qt, tk):
    qi = pl.program_id(1)
    S = k_ref.shape[2]
    D = k_ref.shape[3]
    nb = S // MBLK
    ng = qt // MBLK

    q = q_ref[0, 0]
    mask_slab = m_ref[0].astype(jnp.float32)

    g_rows = jax.lax.broadcasted_iota(jnp.int32, (ng, qt), 0)
    g_cols = jax.lax.broadcasted_iota(jnp.int32, (ng, qt), 1)
    G = (g_cols // MBLK == g_rows).astype(jnp.float32)
    colsum = jax.lax.dot_general(
        G, mask_slab, (((1,), (0,)), ((), ())),
        preferred_element_type=jnp.float32)
    blk_ids = jax.lax.broadcasted_iota(jnp.int32, (nb, S), 0)
    col_ids = jax.lax.broadcasted_iota(jnp.int32, (nb, S), 1)
    E = (col_ids // MBLK == blk_ids).astype(jnp.float32)
    blocksum = jax.lax.dot_general(
        colsum, E, (((1,), (1,)), ((), ())),
        preferred_element_type=jnp.float32)
    active = (blocksum > 0).astype(jnp.float32)
    allow_ref[...] = jax.lax.dot_general(
        active, E, (((1,), (0,)), ((), ())),
        preferred_element_type=jnp.float32)

    row_ids = qi * qt + jax.lax.broadcasted_iota(jnp.int32, (qt, tk), 0)
    col_iota = jax.lax.broadcasted_iota(jnp.int32, (qt, tk), 1)
    n_chunks = qi * qt // tk + 1
    m0 = jnp.full((qt, 1), NEG, jnp.float32)
    l0 = jnp.zeros((qt, 1), jnp.float32)
    acc0 = jnp.zeros((qt, D), jnp.float32)

    def qk(j):
        k = k_ref[0, 0, pl.ds(j * tk, tk), :]
        return jax.lax.dot_general(
            q, k, (((1,), (1,)), ((), ())),
            preferred_element_type=jnp.float32)

    def flash_update(j, s, carry):
        m_i, l_i, acc = carry
        v = v_ref[0, 0, pl.ds(j * tk, tk), :]
        m_new = jnp.maximum(m_i, jnp.max(s, axis=1, keepdims=True))
        p = jnp.exp(s - m_new)
        alpha = jnp.exp(m_i - m_new)
        l_new = l_i * alpha + jnp.sum(p, axis=1, keepdims=True)
        acc_new = acc * alpha + jax.lax.dot_general(
            p, v, (((1,), (0,)), ((), ())),
            preferred_element_type=jnp.float32)
        return m_new, l_new, acc_new

    all_active = jnp.min(blocksum) > 0

    @pl.when(all_active)
    def _fast():
        def body(j, carry):
            return flash_update(j, qk(j), carry)

        carry = jax.lax.fori_loop(0, n_chunks - 1, body, (m0, l0, acc0))
        jd = n_chunks - 1
        s = qk(jd)
        s = jnp.where(jd * tk + col_iota <= row_ids, s, NEG)
        m_f, l_f, acc_f = flash_update(jd, s, carry)
        o_ref[0, 0] = acc_f / l_f

    @pl.when(jnp.logical_not(all_active))
    def _general():
        def body(j, carry):
            m_i, l_i, acc = carry
            s = qk(j)
            v = v_ref[0, 0, pl.ds(j * tk, tk), :]
            allow_g = allow_ref[:, pl.ds(j * tk, tk)]
            ballow = jnp.concatenate(
                [jnp.broadcast_to(allow_g[g:g + 1, :], (MBLK, tk))
                 for g in range(ng)], axis=0) > 0.5
            allow = ballow & (j * tk + col_iota <= row_ids)
            s = jnp.where(allow, s, NEG)
            m_new = jnp.maximum(m_i, jnp.max(s, axis=1, keepdims=True))
            p = jnp.exp(s - m_new) * allow.astype(jnp.float32)
            alpha = jnp.exp(m_i - m_new)
            l_new = l_i * alpha + jnp.sum(p, axis=1, keepdims=True)
            acc_new = acc * alpha + jax.lax.dot_general(
                p, v, (((1,), (0,)), ((), ())),
                preferred_element_type=jnp.float32)
            return m_new, l_new, acc_new

        m_f, l_f, acc_f = jax.lax.fori_loop(0, n_chunks, body, (m0, l0, acc0))
        o_ref[0, 0] = jnp.where(l_f > 0, acc_f / jnp.maximum(l_f, 1e-30), 0.0)


@jax.jit
def kernel(query, key, value, mask):
    B, H, S, D = query.shape
    qt = min(1024, S)
    tk = min(1024, S)
    ng = qt // MBLK
    grid = (H, S // qt)
    out = pl.pallas_call(
        functools.partial(_attn_kernel, qt=qt, tk=tk),
        grid=grid,
        in_specs=[
            pl.BlockSpec((1, 1, qt, D), lambda h, i: (0, h, i, 0)),
            pl.BlockSpec((1, 1, S, D), lambda h, i: (0, h, 0, 0)),
            pl.BlockSpec((1, 1, S, D), lambda h, i: (0, h, 0, 0)),
            pl.BlockSpec((1, qt, S), lambda h, i: (h, i, 0)),
        ],
        out_specs=pl.BlockSpec((1, 1, qt, D), lambda h, i: (0, h, i, 0)),
        out_shape=jax.ShapeDtypeStruct((B, H, S, D), jnp.float32),
        scratch_shapes=[pltpu.VMEM((ng, S), jnp.float32)],
        compiler_params=pltpu.CompilerParams(
            dimension_semantics=("parallel", "arbitrary")),
    )(query, key, value, mask)
    return out
